# Initial kernel scaffold; baseline (speedup 1.0000x reference)
import jax
import jax.numpy as jnp
from jax.experimental import pallas as pl


def kernel(node, rel, edge_index, edge_type, fre, norm, w_triplet, w_quad, loop_weight, evolve_loop_weight):
    raise NotImplementedError("write your pallas kernel here")


# speedup vs baseline: 1.2427x; 1.2427x over previous
import functools

import jax
import jax.numpy as jnp
from jax import lax
from jax.experimental import pallas as pl
from jax.experimental.pallas import tpu as pltpu
from jax.experimental.pallas import tpu_sc as plsc

F = 256
FW = 512
NC, NS, L = 2, 16, 16
NT = NC * NS
PH = 2
OWN = 160
B = 16
FB = 128
SBL = 416
TRASH = 400
SC_CHUNK = 256
RB = 10


def _dense_body(node_ref, rel_ref, wt_ref, wq_ref, lw_ref, elw_ref,
                asrc_ref, adst_ref, arel_ref, csum_ref, l_ref, el_ref):
    wq = wq_ref[...]
    nb = node_ref[...]
    w1 = wt_ref[0:F, :]
    w2 = wt_ref[F:2 * F, :]
    w3 = wt_ref[2 * F:3 * F, :]
    p1 = jnp.dot(nb, w1, preferred_element_type=jnp.float32)
    asrc_ref[:, 0:F] = p1
    asrc_ref[:, F:FW] = jnp.dot(p1, wq, preferred_element_type=jnp.float32)
    p3 = jnp.dot(nb, w3, preferred_element_type=jnp.float32)
    adst_ref[:, 0:F] = p3
    adst_ref[:, F:FW] = jnp.dot(p3, wq, preferred_element_type=jnp.float32)
    pr = jnp.dot(rel_ref[...], w2, preferred_element_type=jnp.float32)
    arel_ref[:, 0:F] = pr
    arel_ref[:, F:FW] = jnp.dot(pr, wq, preferred_element_type=jnp.float32)
    csum_ref[...] = jnp.sum(wq, axis=0, keepdims=True)
    l_ref[...] = jnp.dot(nb, lw_ref[...], preferred_element_type=jnp.float32)
    el_ref[...] = jnp.dot(nb, elw_ref[...], preferred_element_type=jnp.float32)


def _final_body(nd_ref, norm_ref, l_ref, el_ref, h_ref):
    nd = nd_ref[...]
    num = nd[:, 0:F]
    den = nd[:, F:FW]
    agg = jnp.where(den > 0, num / jnp.maximum(den, 1e-30), 0.0)
    loop = jnp.where(den[:, 0:1] > 0, l_ref[...], el_ref[...])
    h_ref[...] = agg * norm_ref[...] + loop


def _make_edge_kernel(n_scan, cap):
    mesh = plsc.VectorSubcoreMesh(
        core_axis_name="c", subcore_axis_name="s",
        num_cores=NC, num_subcores=NS)

    @functools.partial(
        pl.kernel,
        out_type=(
            jax.ShapeDtypeStruct((NT * PH * OWN, FW), jnp.float32),
            jax.ShapeDtypeStruct((NT * PH, cap), jnp.int32),
        ),
        mesh=mesh,
        compiler_params=pltpu.CompilerParams(needs_layout_passes=False),
        scratch_types=[
            pltpu.VMEM((SC_CHUNK,), jnp.int32),
            pltpu.VMEM((SBL,), jnp.int32),
            pltpu.VMEM((48,), jnp.int32),
            pltpu.VMEM((FB,), jnp.int32),
            pltpu.VMEM((B,), jnp.int32),
            pltpu.VMEM((B,), jnp.int32),
            pltpu.VMEM((B,), jnp.int32),
            pltpu.VMEM((B + L,), jnp.int32),
            pltpu.VMEM((B + L,), jnp.float32),
            pltpu.VMEM((B + L,), jnp.float32),
            pltpu.VMEM((B, FW), jnp.float32),
            pltpu.VMEM((B, FW), jnp.float32),
            pltpu.VMEM((B, FW), jnp.float32),
            pltpu.VMEM((F,), jnp.float32),
            pltpu.VMEM((OWN, FW), jnp.float32),
            pltpu.SemaphoreType.DMA,
            pltpu.SemaphoreType.DMA,
            pltpu.SemaphoreType.DMA,
            pltpu.SemaphoreType.DMA,
        ],
    )
    def edge_kernel(asrc, adst, arel, csum_hbm, src_h, dst_h, ty_h, fre_h,
                    out, idh,
                    scan_v, sb, pbuf, idb, src_v, dst_v, ty_v,
                    lidx_v, fre_v, mrow_v, b1, b2, b3, csum_v, acc,
                    semr, sem1, sem2, sem3):
        c = lax.axis_index("c")
        s = lax.axis_index("s")
        wid = c * NS + s
        e_dummy = src_h.shape[0] - 1
        pltpu.sync_copy(csum_hbm, csum_v)
        iota = lax.iota(jnp.int32, L)
        zero = jnp.zeros((L,), jnp.float32)
        izero = jnp.zeros((L,), jnp.int32)
        pbuf[pl.ds(0, L)] = izero

        def phase_body(p, ph_carry):
            tile_lo = (p * NT + wid) * OWN
            tp = wid * PH + p

            def zero_body(r, zc):
                for g in range(FW // L):
                    acc[r, pl.ds(g * L, L)] = zero
                return zc

            lax.fori_loop(0, OWN, zero_body, 0)

            def flush_if_full(soff, nf):
                do = soff >= FB

                @pl.when(do)
                def _():
                    pltpu.sync_copy(sb.at[pl.ds(0, FB)],
                                    idh.at[tp, pl.ds(nf * FB, FB)])
                    for j in range(16):
                        sb[pl.ds(j * L, L)] = sb[pl.ds(FB + j * L, L)]

                soff = jnp.where(do, soff - FB, soff)
                nf = jnp.where(do, nf + 1, nf)
                return soff, nf

            def scan_body(k, carry):
                soff, nf = carry
                pltpu.sync_copy(dst_h.at[pl.ds(k * SC_CHUNK, SC_CHUNK)],
                                scan_v)
                for i in range(SC_CHUNK // L):
                    dv = scan_v[pl.ds(i * L, L)]
                    lv = dv - tile_lo
                    m = (lv >= 0) & (lv < OWN)
                    idv = iota + (k * SC_CHUNK + i * L)
                    pr_ = jnp.where(m, 1, 0)
                    for sh in (1, 2, 4, 8):
                        pbuf[pl.ds(L, L)] = pr_
                        pr_ = pr_ + pbuf[pl.ds(L - sh, L)]
                    pbuf[pl.ds(L, L)] = pr_
                    total = pbuf[pl.ds(2 * L - 1, L)][0]
                    pos = jnp.where(m, soff + pr_ - 1, TRASH + iota)
                    plsc.store_scatter(sb, [pos], idv)
                    soff = soff + total
                soff, nf = flush_if_full(soff, nf)
                soff, nf = flush_if_full(soff, nf)
                return soff, nf

            soff, nf = lax.fori_loop(0, n_scan, scan_body,
                                     (jnp.int32(0), jnp.int32(0)))
            dummy = jnp.full((L,), e_dummy, jnp.int32)
            for j in range(FB // L):
                sb[pl.ds(soff + j * L, L)] = dummy
            pltpu.sync_copy(sb.at[pl.ds(0, FB)],
                            idh.at[tp, pl.ds(nf * FB, FB)])
            trips = nf + 1

            def block_cond(bi):
                return bi < trips

            def block_body(bi):
                pltpu.sync_copy(idh.at[tp, pl.ds(bi * FB, FB)], idb)
                for j in range(FB // B):
                    idsl = idb.at[pl.ds(j * B, B)]
                    cp0 = pltpu.async_copy(src_h.at[idsl], src_v, semr)
                    cp1 = pltpu.async_copy(dst_h.at[idsl], dst_v, sem1)
                    cp2 = pltpu.async_copy(ty_h.at[idsl], ty_v, sem2)
                    cp3 = pltpu.async_copy(fre_h.at[idsl],
                                           fre_v.at[pl.ds(0, B)], sem3)
                    cp0.wait()
                    cp1.wait()
                    cp2.wait()
                    cp3.wait()
                    dstv = dst_v[pl.ds(0, L)]
                    lv = dstv - tile_lo
                    m = (lv >= 0) & (lv < OWN)
                    lidx_v[pl.ds(0, L)] = jnp.where(m, lv, 0)
                    mrow_v[pl.ds(0, L)] = jnp.where(m, 1.0, 0.0)
                    cg1 = pltpu.async_copy(asrc.at[src_v], b1, sem1)
                    cg2 = pltpu.async_copy(adst.at[dst_v], b2, sem2)
                    cg3 = pltpu.async_copy(arel.at[ty_v], b3, sem3)
                    cg1.wait()
                    cg2.wait()
                    cg3.wait()

                    def row_body(r, rc):
                        rl = lidx_v[pl.ds(r, L)][0]
                        fb_ = zero + fre_v[pl.ds(r, L)][0]
                        mb = zero + mrow_v[pl.ds(r, L)][0]
                        for g in range(F // L):
                            ca = F + g * L
                            ct = g * L
                            a = (b1[r, pl.ds(ca, L)] + b2[r, pl.ds(ca, L)]
                                 + b3[r, pl.ds(ca, L)]
                                 + fb_ * csum_v[pl.ds(ct, L)])
                            w = jnp.exp(jnp.maximum(a, 0.01 * a)) * mb
                            t = (b1[r, pl.ds(ct, L)] + b2[r, pl.ds(ct, L)]
                                 + b3[r, pl.ds(ct, L)]) * w
                            acc[rl, pl.ds(ct, L)] = acc[rl, pl.ds(ct, L)] + t
                            acc[rl, pl.ds(ca, L)] = acc[rl, pl.ds(ca, L)] + w
                        return rc

                    lax.fori_loop(0, B, row_body, 0)
                return bi + 1

            lax.while_loop(block_cond, block_body, jnp.int32(0))

            pltpu.sync_copy(acc, out.at[pl.ds(tile_lo, OWN)])
            return ph_carry

        lax.fori_loop(0, PH, phase_body, 0)

    return edge_kernel


def kernel(node, rel, edge_index, edge_type, fre, norm,
           w_triplet, w_quad, loop_weight, evolve_loop_weight):
    n = node.shape[0]
    e = edge_index.shape[1]
    rblk = n // RB

    asrc, adst, arel, csum, lmat, elmat = pl.pallas_call(
        _dense_body,
        grid=(RB,),
        in_specs=[
            pl.BlockSpec((rblk, F), lambda i: (i, 0)),
            pl.BlockSpec(rel.shape, lambda i: (0, 0)),
            pl.BlockSpec((3 * F, F), lambda i: (0, 0)),
            pl.BlockSpec((F, F), lambda i: (0, 0)),
            pl.BlockSpec((F, F), lambda i: (0, 0)),
            pl.BlockSpec((F, F), lambda i: (0, 0)),
        ],
        out_specs=[
            pl.BlockSpec((rblk, FW), lambda i: (i, 0)),
            pl.BlockSpec((rblk, FW), lambda i: (i, 0)),
            pl.BlockSpec((rel.shape[0], FW), lambda i: (0, 0)),
            pl.BlockSpec((1, F), lambda i: (0, 0)),
            pl.BlockSpec((rblk, F), lambda i: (i, 0)),
            pl.BlockSpec((rblk, F), lambda i: (i, 0)),
        ],
        out_shape=[
            jax.ShapeDtypeStruct((n, FW), jnp.float32),
            jax.ShapeDtypeStruct((n, FW), jnp.float32),
            jax.ShapeDtypeStruct((rel.shape[0], FW), jnp.float32),
            jax.ShapeDtypeStruct((1, F), jnp.float32),
            jax.ShapeDtypeStruct((n, F), jnp.float32),
            jax.ShapeDtypeStruct((n, F), jnp.float32),
        ],
    )(node, rel, w_triplet, w_quad, loop_weight, evolve_loop_weight)

    sentinel = jnp.int32(NT * PH * OWN + 7)
    e_pad = e + SC_CHUNK
    src_p = jnp.concatenate([edge_index[0], jnp.zeros((e_pad - e,), jnp.int32)])
    dst_p = jnp.concatenate([edge_index[1],
                             jnp.full((e_pad - e,), sentinel, jnp.int32)])
    ty_p = jnp.concatenate([edge_type, jnp.zeros((e_pad - e,), jnp.int32)])
    fre_p = jnp.concatenate([fre, jnp.zeros((e_pad - e,), jnp.float32)])
    cap = (e // FB + 2) * FB

    nd, _ = _make_edge_kernel(e_pad // SC_CHUNK, cap)(
        asrc, adst, arel, csum.reshape(F), src_p, dst_p, ty_p, fre_p)
    nd = nd[:n]

    h = pl.pallas_call(
        _final_body,
        grid=(RB,),
        in_specs=[
            pl.BlockSpec((rblk, FW), lambda i: (i, 0)),
            pl.BlockSpec((rblk, 1), lambda i: (i, 0)),
            pl.BlockSpec((rblk, F), lambda i: (i, 0)),
            pl.BlockSpec((rblk, F), lambda i: (i, 0)),
        ],
        out_specs=pl.BlockSpec((rblk, F), lambda i: (i, 0)),
        out_shape=jax.ShapeDtypeStruct((n, F), jnp.float32),
    )(nd, norm, lmat, elmat)
    return h


# speedup vs baseline: 1.3914x; 1.1196x over previous
import functools

import jax
import jax.numpy as jnp
from jax import lax
from jax.experimental import pallas as pl
from jax.experimental.pallas import tpu as pltpu
from jax.experimental.pallas import tpu_sc as plsc

F = 256
FW = 512
NC, NS, L = 2, 16, 16
NT = NC * NS
PH = 2
OWN = 160
B = 16
FB = 128
SBL = 416
TRASH = 400
SC_CHUNK = 256
RB = 10


def _dense_body(node_ref, rel_ref, wt_ref, wq_ref, lw_ref, elw_ref,
                asrc_ref, adst_ref, arel_ref, csum_ref, l_ref, el_ref):
    wq = wq_ref[...]
    nb = node_ref[...]
    w1 = wt_ref[0:F, :]
    w2 = wt_ref[F:2 * F, :]
    w3 = wt_ref[2 * F:3 * F, :]
    p1 = jnp.dot(nb, w1, preferred_element_type=jnp.float32)
    asrc_ref[:, 0:F] = p1
    asrc_ref[:, F:FW] = jnp.dot(p1, wq, preferred_element_type=jnp.float32)
    p3 = jnp.dot(nb, w3, preferred_element_type=jnp.float32)
    adst_ref[:, 0:F] = p3
    adst_ref[:, F:FW] = jnp.dot(p3, wq, preferred_element_type=jnp.float32)
    pr = jnp.dot(rel_ref[...], w2, preferred_element_type=jnp.float32)
    arel_ref[:, 0:F] = pr
    arel_ref[:, F:FW] = jnp.dot(pr, wq, preferred_element_type=jnp.float32)
    csum_ref[...] = jnp.sum(wq, axis=0, keepdims=True)
    l_ref[...] = jnp.dot(nb, lw_ref[...], preferred_element_type=jnp.float32)
    el_ref[...] = jnp.dot(nb, elw_ref[...], preferred_element_type=jnp.float32)


def _final_body(nd_ref, norm_ref, l_ref, el_ref, h_ref):
    nd = nd_ref[...]
    num = nd[:, 0:F]
    den = nd[:, F:FW]
    agg = jnp.where(den > 0, num / jnp.maximum(den, 1e-30), 0.0)
    loop = jnp.where(den[:, 0:1] > 0, l_ref[...], el_ref[...])
    h_ref[...] = agg * norm_ref[...] + loop


def _make_edge_kernel(n_scan, cap):
    mesh = plsc.VectorSubcoreMesh(
        core_axis_name="c", subcore_axis_name="s",
        num_cores=NC, num_subcores=NS)

    @functools.partial(
        pl.kernel,
        out_type=(
            jax.ShapeDtypeStruct((NT * PH * OWN, FW), jnp.float32),
            jax.ShapeDtypeStruct((NT * PH, cap), jnp.int32),
        ),
        mesh=mesh,
        compiler_params=pltpu.CompilerParams(needs_layout_passes=False),
        scratch_types=[
            pltpu.VMEM((SC_CHUNK,), jnp.int32),
            pltpu.VMEM((SBL,), jnp.int32),
            pltpu.VMEM((48,), jnp.int32),
            pltpu.VMEM((FB,), jnp.int32),
            pltpu.VMEM((B,), jnp.int32),
            pltpu.VMEM((B,), jnp.int32),
            pltpu.VMEM((B,), jnp.int32),
            pltpu.VMEM((B + L,), jnp.int32),
            pltpu.VMEM((B + L,), jnp.float32),
            pltpu.VMEM((B + L,), jnp.float32),
            pltpu.VMEM((B, FW), jnp.float32),
            pltpu.VMEM((B, FW), jnp.float32),
            pltpu.VMEM((B, FW), jnp.float32),
            pltpu.VMEM((F,), jnp.float32),
            pltpu.VMEM((OWN, FW), jnp.float32),
            pltpu.SemaphoreType.DMA,
            pltpu.SemaphoreType.DMA,
            pltpu.SemaphoreType.DMA,
            pltpu.SemaphoreType.DMA,
        ],
    )
    def edge_kernel(asrc, adst, arel, csum_hbm, src_h, dst_h, ty_h, fre_h,
                    out, idh,
                    scan_v, sb, pbuf, idb, src_v, dst_v, ty_v,
                    lidx_v, fre_v, mrow_v, b1, b2, b3, csum_v, acc,
                    semr, sem1, sem2, sem3):
        c = lax.axis_index("c")
        s = lax.axis_index("s")
        wid = c * NS + s
        e_dummy = src_h.shape[0] - 1
        pltpu.sync_copy(csum_hbm, csum_v)
        iota = lax.iota(jnp.int32, L)
        zero = jnp.zeros((L,), jnp.float32)
        izero = jnp.zeros((L,), jnp.int32)
        pbuf[pl.ds(0, L)] = izero

        def phase_body(p, ph_carry):
            tile_lo = (p * NT + wid) * OWN
            tp = wid * PH + p

            def zero_body(r, zc):
                for g in range(FW // L):
                    acc[r, pl.ds(g * L, L)] = zero
                return zc

            lax.fori_loop(0, OWN, zero_body, 0)

            def flush_if_full(soff, nf):
                do = soff >= FB

                @pl.when(do)
                def _():
                    pltpu.sync_copy(sb.at[pl.ds(0, FB)],
                                    idh.at[tp, pl.ds(nf * FB, FB)])
                    for j in range(16):
                        sb[pl.ds(j * L, L)] = sb[pl.ds(FB + j * L, L)]

                soff = jnp.where(do, soff - FB, soff)
                nf = jnp.where(do, nf + 1, nf)
                return soff, nf

            def scan_body(k, carry):
                soff, nf = carry
                pltpu.sync_copy(dst_h.at[pl.ds(k * SC_CHUNK, SC_CHUNK)],
                                scan_v)
                for i in range(SC_CHUNK // L):
                    dv = scan_v[pl.ds(i * L, L)]
                    lv = dv - tile_lo
                    m = (lv >= 0) & (lv < OWN)
                    idv = iota + (k * SC_CHUNK + i * L)
                    plsc.store_compressed(sb.at[pl.ds(soff, L)], idv, mask=m)
                    soff = soff + plsc.all_reduce_population_count(m)[0]
                soff, nf = flush_if_full(soff, nf)
                soff, nf = flush_if_full(soff, nf)
                return soff, nf

            soff, nf = lax.fori_loop(0, n_scan, scan_body,
                                     (jnp.int32(0), jnp.int32(0)))
            dummy = jnp.full((L,), e_dummy, jnp.int32)
            for j in range(FB // L):
                sb[pl.ds(soff + j * L, L)] = dummy
            pltpu.sync_copy(sb.at[pl.ds(0, FB)],
                            idh.at[tp, pl.ds(nf * FB, FB)])
            trips = nf + 1

            def block_cond(bi):
                return bi < trips

            def block_body(bi):
                pltpu.sync_copy(idh.at[tp, pl.ds(bi * FB, FB)], idb)
                for j in range(FB // B):
                    idsl = idb.at[pl.ds(j * B, B)]
                    cp0 = pltpu.async_copy(src_h.at[idsl], src_v, semr)
                    cp1 = pltpu.async_copy(dst_h.at[idsl], dst_v, sem1)
                    cp2 = pltpu.async_copy(ty_h.at[idsl], ty_v, sem2)
                    cp3 = pltpu.async_copy(fre_h.at[idsl],
                                           fre_v.at[pl.ds(0, B)], sem3)
                    cp0.wait()
                    cp1.wait()
                    cp2.wait()
                    cp3.wait()
                    dstv = dst_v[pl.ds(0, L)]
                    lv = dstv - tile_lo
                    m = (lv >= 0) & (lv < OWN)
                    lidx_v[pl.ds(0, L)] = jnp.where(m, lv, 0)
                    mrow_v[pl.ds(0, L)] = jnp.where(m, 1.0, 0.0)
                    cg1 = pltpu.async_copy(asrc.at[src_v], b1, sem1)
                    cg2 = pltpu.async_copy(adst.at[dst_v], b2, sem2)
                    cg3 = pltpu.async_copy(arel.at[ty_v], b3, sem3)
                    cg1.wait()
                    cg2.wait()
                    cg3.wait()

                    def row_body(r, rc):
                        rl = lidx_v[pl.ds(r, L)][0]
                        fb_ = zero + fre_v[pl.ds(r, L)][0]
                        mb = zero + mrow_v[pl.ds(r, L)][0]
                        for g in range(F // L):
                            ca = F + g * L
                            ct = g * L
                            a = (b1[r, pl.ds(ca, L)] + b2[r, pl.ds(ca, L)]
                                 + b3[r, pl.ds(ca, L)]
                                 + fb_ * csum_v[pl.ds(ct, L)])
                            w = jnp.exp(jnp.maximum(a, 0.01 * a)) * mb
                            t = (b1[r, pl.ds(ct, L)] + b2[r, pl.ds(ct, L)]
                                 + b3[r, pl.ds(ct, L)]) * w
                            plsc.addupdate(acc.at[rl, pl.ds(ct, L)], t)
                            plsc.addupdate(acc.at[rl, pl.ds(ca, L)], w)
                        return rc

                    lax.fori_loop(0, B, row_body, 0)
                return bi + 1

            lax.while_loop(block_cond, block_body, jnp.int32(0))

            pltpu.sync_copy(acc, out.at[pl.ds(tile_lo, OWN)])
            return ph_carry

        lax.fori_loop(0, PH, phase_body, 0)

    return edge_kernel


def kernel(node, rel, edge_index, edge_type, fre, norm,
           w_triplet, w_quad, loop_weight, evolve_loop_weight):
    n = node.shape[0]
    e = edge_index.shape[1]
    rblk = n // RB

    asrc, adst, arel, csum, lmat, elmat = pl.pallas_call(
        _dense_body,
        grid=(RB,),
        in_specs=[
            pl.BlockSpec((rblk, F), lambda i: (i, 0)),
            pl.BlockSpec(rel.shape, lambda i: (0, 0)),
            pl.BlockSpec((3 * F, F), lambda i: (0, 0)),
            pl.BlockSpec((F, F), lambda i: (0, 0)),
            pl.BlockSpec((F, F), lambda i: (0, 0)),
            pl.BlockSpec((F, F), lambda i: (0, 0)),
        ],
        out_specs=[
            pl.BlockSpec((rblk, FW), lambda i: (i, 0)),
            pl.BlockSpec((rblk, FW), lambda i: (i, 0)),
            pl.BlockSpec((rel.shape[0], FW), lambda i: (0, 0)),
            pl.BlockSpec((1, F), lambda i: (0, 0)),
            pl.BlockSpec((rblk, F), lambda i: (i, 0)),
            pl.BlockSpec((rblk, F), lambda i: (i, 0)),
        ],
        out_shape=[
            jax.ShapeDtypeStruct((n, FW), jnp.float32),
            jax.ShapeDtypeStruct((n, FW), jnp.float32),
            jax.ShapeDtypeStruct((rel.shape[0], FW), jnp.float32),
            jax.ShapeDtypeStruct((1, F), jnp.float32),
            jax.ShapeDtypeStruct((n, F), jnp.float32),
            jax.ShapeDtypeStruct((n, F), jnp.float32),
        ],
    )(node, rel, w_triplet, w_quad, loop_weight, evolve_loop_weight)

    sentinel = jnp.int32(NT * PH * OWN + 7)
    e_pad = e + SC_CHUNK
    src_p = jnp.concatenate([edge_index[0], jnp.zeros((e_pad - e,), jnp.int32)])
    dst_p = jnp.concatenate([edge_index[1],
                             jnp.full((e_pad - e,), sentinel, jnp.int32)])
    ty_p = jnp.concatenate([edge_type, jnp.zeros((e_pad - e,), jnp.int32)])
    fre_p = jnp.concatenate([fre, jnp.zeros((e_pad - e,), jnp.float32)])
    cap = (e // FB + 2) * FB

    nd, _ = _make_edge_kernel(e_pad // SC_CHUNK, cap)(
        asrc, adst, arel, csum.reshape(F), src_p, dst_p, ty_p, fre_p)
    nd = nd[:n]

    h = pl.pallas_call(
        _final_body,
        grid=(RB,),
        in_specs=[
            pl.BlockSpec((rblk, FW), lambda i: (i, 0)),
            pl.BlockSpec((rblk, 1), lambda i: (i, 0)),
            pl.BlockSpec((rblk, F), lambda i: (i, 0)),
            pl.BlockSpec((rblk, F), lambda i: (i, 0)),
        ],
        out_specs=pl.BlockSpec((rblk, F), lambda i: (i, 0)),
        out_shape=jax.ShapeDtypeStruct((n, F), jnp.float32),
    )(nd, norm, lmat, elmat)
    return h
